# Initial kernel scaffold; baseline (speedup 1.0000x reference)
import jax
import jax.numpy as jnp
from jax.experimental import pallas as pl


def kernel(V_embed, E_embed, X, x_mask, chain_idx, params):
    raise NotImplementedError("write your pallas kernel here")


# speedup vs baseline: 5.1805x; 5.1805x over previous
import math
import jax
import jax.numpy as jnp
from jax import lax
from jax.experimental import pallas as pl
from jax.experimental.pallas import tpu as pltpu
from jax.experimental.pallas import tpu_sc as plsc

B, N, K, H = 8, 256, 30, 128
EIN = 128
OUT = 400
NUM_LAYERS = 3
E = N * K
NT = 4
ET = E // NT
NPT = N // NT

_NC, _NS = 2, 16
_NW = _NC * _NS
_ROWS_PER_W = (B * E) // _NW
_CHUNK = 128
_NCHUNK = _ROWS_PER_W // _CHUNK


def _ln(x, g, b, eps=1e-5):
    mu = jnp.mean(x, axis=-1, keepdims=True)
    var = jnp.mean((x - mu) ** 2, axis=-1, keepdims=True)
    return (x - mu) / jnp.sqrt(var + eps) * g + b


def _dot(a, b):
    return jnp.dot(a, b, preferred_element_type=jnp.float32)


def _full(shape):
    return pl.BlockSpec(shape, lambda *_: (0,) * len(shape))


def _batched(shape):
    return pl.BlockSpec((1,) + shape, lambda b, *_: (b,) + (0,) * len(shape))


def _tiled(shape):
    return pl.BlockSpec((1,) + shape, lambda b, t: (b, t) + (0,) * (len(shape) - 1))


def _feat_kernel(ca_ref, cat_ref, xb_ref, mcol_ref, mrow_ref, vembed_ref,
                 wvf_ref, bvf_ref, lnvg_ref, lnvb_ref, wv_ref, bv_ref,
                 eidx_ref, dnb_ref, gidx_ref, hv0_ref):
    b = pl.program_id(0)
    ca = ca_ref[0]
    cat = cat_ref[0]
    mcol = mcol_ref[0]
    mrow = mrow_ref[0]

    d2 = jnp.zeros((N, N), jnp.float32)
    for c in range(3):
        diff = ca[:, c:c + 1] - cat[c:c + 1, :]
        d2 = d2 + diff * diff
    dist = jnp.sqrt(d2 + 1e-6)
    mask2 = mcol * mrow
    work = dist + (1.0 - mask2) * 1e6

    lane = lax.broadcasted_iota(jnp.int32, (N, N), 1)
    idx_cols = []
    val_cols = []
    for _ in range(K):
        m = jnp.min(work, axis=1, keepdims=True)
        hit = work == m
        idx = jnp.min(jnp.where(hit, lane, N), axis=1, keepdims=True)
        idx_cols.append(idx)
        val_cols.append(m)
        work = jnp.where(lane == idx, 1e30, work)
    eidx = jnp.concatenate(idx_cols, axis=1)
    dnb = jnp.concatenate(val_cols, axis=1)
    eidx_ref[0] = eidx
    dnb_ref[0] = dnb
    gidx_ref[0] = eidx + (b * N + lax.broadcasted_iota(jnp.int32, (N, K), 0)) * N

    xb = xb_ref[0]
    dx = xb[1:, :] - xb[:-1, :]
    nrm = jnp.sqrt(jnp.sum(dx * dx, axis=1, keepdims=True))
    u = dx / (nrm + 1e-7)
    u2 = u[:-2, :]
    u1 = u[1:-1, :]
    u0 = u[2:, :]

    def cross(a, bb):
        return jnp.concatenate([
            a[:, 1:2] * bb[:, 2:3] - a[:, 2:3] * bb[:, 1:2],
            a[:, 2:3] * bb[:, 0:1] - a[:, 0:1] * bb[:, 2:3],
            a[:, 0:1] * bb[:, 1:2] - a[:, 1:2] * bb[:, 0:1],
        ], axis=1)

    def norm3(v):
        nn = jnp.sqrt(jnp.sum(v * v, axis=1, keepdims=True))
        return v / (nn + 1e-7)

    n2 = norm3(cross(u2, u1))
    n1 = norm3(cross(u1, u0))
    cosd = jnp.sum(n2 * n1, axis=1, keepdims=True)
    cosd = jnp.clip(cosd, -1.0 + 1e-7, 1.0 - 1e-7)
    sgn = jnp.sign(jnp.sum(u2 * n1, axis=1, keepdims=True))
    sind = sgn * jnp.sqrt(jnp.maximum(1.0 - cosd * cosd, 0.0))
    cosp = jnp.concatenate(
        [jnp.ones((1, 1), jnp.float32), cosd, jnp.ones((2, 1), jnp.float32)], axis=0)
    sinp = jnp.concatenate(
        [jnp.zeros((1, 1), jnp.float32), sind, jnp.zeros((2, 1), jnp.float32)], axis=0)

    i_row = lax.broadcasted_iota(jnp.int32, (N, 3 * N), 0)
    j_col = lax.broadcasted_iota(jnp.int32, (N, 3 * N), 1)
    cols = []
    for c in range(3):
        sel = jnp.where(j_col == 3 * i_row + c, 1.0, 0.0)
        cols.append(_dot(sel, cosp))
    for c in range(3):
        sel = jnp.where(j_col == 3 * i_row + c, 1.0, 0.0)
        cols.append(_dot(sel, sinp))
    vraw = jnp.concatenate(cols, axis=1)

    v = _ln(_dot(vraw, wvf_ref[...]) + bvf_ref[...], lnvg_ref[...], lnvb_ref[...])
    wv = wv_ref[...]
    hv0_ref[0] = (_dot(v, wv[:H, :]) + _dot(vembed_ref[0], wv[H:, :])
                  + bv_ref[...])


def _run_features(Ca, CaT, Xb, mcol, mrow, V_embed, params):
    out_shapes = (
        jax.ShapeDtypeStruct((B, N, K), jnp.int32),
        jax.ShapeDtypeStruct((B, N, K), jnp.float32),
        jax.ShapeDtypeStruct((B, N, K), jnp.int32),
        jax.ShapeDtypeStruct((B, N, H), jnp.float32),
    )
    return pl.pallas_call(
        _feat_kernel,
        grid=(B,),
        in_specs=[
            _batched((N, 3)), _batched((3, N)), _batched((3 * N, 3)),
            _batched((N, 1)), _batched((1, N)), _batched((N, EIN)),
            _full((6, H)), _full((1, H)), _full((1, H)), _full((1, H)),
            _full((H + EIN, H)), _full((1, H)),
        ],
        out_specs=[_batched((N, K)), _batched((N, K)), _batched((N, K)),
                   _batched((N, H))],
        out_shape=out_shapes,
    )(Ca, CaT, Xb, mcol, mrow, V_embed,
      params['Wvf'], params['bvf'].reshape(1, H),
      params['ln_vf_g'].reshape(1, H), params['ln_vf_b'].reshape(1, H),
      params['Wv'], params['bv'].reshape(1, H))


def _sc_gather_body(table_hbm, idx_hbm, out_hbm, idx_v, buf_v, sem):
    wid = lax.axis_index("s") * _NC + lax.axis_index("c")
    base = wid * _ROWS_PER_W
    pltpu.sync_copy(idx_hbm.at[pl.ds(base, _ROWS_PER_W)], idx_v)
    for j in range(_NCHUNK):
        pltpu.async_copy(
            table_hbm.at[idx_v.at[pl.ds(j * _CHUNK, _CHUNK)]], buf_v, sem).wait()
        pltpu.sync_copy(buf_v, out_hbm.at[pl.ds(base + j * _CHUNK, _CHUNK)])


def _run_sc_gather(table_flat, gidx_flat):
    import functools
    mesh = plsc.VectorSubcoreMesh(core_axis_name="c", subcore_axis_name="s")
    k = functools.partial(
        pl.kernel,
        mesh=mesh,
        out_type=jax.ShapeDtypeStruct((B * E, EIN), jnp.float32),
        scratch_types=[
            pltpu.VMEM((_ROWS_PER_W,), jnp.int32),
            pltpu.VMEM((_CHUNK, EIN), jnp.float32),
            pltpu.SemaphoreType.DMA,
        ],
    )(_sc_gather_body)
    return k(table_flat, gidx_flat)


def _edge_feat_kernel(sc_ref, enb_ref, mcol_ref, ccol_ref, mcolt_ref, ccolt_ref,
                      wef_ref, bef_ref, lneg_ref, lneb_ref, we_ref, be_ref,
                      he0_ref, aux_ref):
    t = pl.program_id(1)
    eidx_f = sc_ref[0][:, 0:1]
    dnb = sc_ref[0][:, 1:2]
    mcol = mcol_ref[0]
    ccol = ccol_ref[0]

    lane_n = lax.broadcasted_iota(jnp.int32, (ET, N), 1).astype(jnp.float32)
    oh = jnp.where(lane_n == eidx_f, 1.0, 0.0)
    e_iota = lax.broadcasted_iota(jnp.int32, (ET, NPT), 0)
    i_iota = lax.broadcasted_iota(jnp.int32, (ET, NPT), 1)
    dd = e_iota - K * i_iota
    ohi = jnp.where((dd >= 0) & (dd < K), 1.0, 0.0)

    node0 = t * NPT
    iota_np = lax.broadcasted_iota(jnp.int32, (NPT, 1), 0).astype(jnp.float32)
    row_i = _dot(ohi, iota_np) + lax.convert_element_type(node0, jnp.float32)
    mcol_t = mcolt_ref[0]
    ccol_t = ccolt_ref[0]
    mask_i = _dot(ohi, mcol_t)
    mask_j = _dot(oh, mcol)
    mask_e = mask_i * mask_j
    ci_i = _dot(ohi, ccol_t)
    ci_j = _dot(oh, ccol)
    same = jnp.where(ci_i == ci_j, 1.0, 0.0)
    rel = (eidx_f - row_i) * same

    kf = lax.broadcasted_iota(jnp.int32, (ET, 16), 1).astype(jnp.float32)
    freq = jnp.exp(kf * 2.0 * (-math.log(10000.0) / 32.0))
    ang = rel * freq
    mu = 2.0 + kf * (20.0 / 15.0)
    sigma = (22.0 - 2.0) / 16.0
    rbf = jnp.exp(-(((dnb - mu) / sigma) ** 2))
    wef = wef_ref[...]
    pre = (_dot(jnp.cos(ang), wef[0:16, :]) + _dot(jnp.sin(ang), wef[16:32, :])
           + _dot(rbf, wef[32:48, :]) + same * wef[48:49, :] + bef_ref[...])
    e_emb = _ln(pre, lneg_ref[...], lneb_ref[...])
    we = we_ref[...]
    he0_ref[0] = _dot(e_emb, we[:H, :]) + _dot(enb_ref[0], we[H:, :]) + be_ref[...]

    aux = jnp.concatenate(
        [eidx_f, mask_e, row_i,
         jnp.zeros((ET, 5), jnp.float32)], axis=1)
    aux_ref[0] = aux


def _run_edge_features(scalars, enb, mcol, ccol, params):
    out_shapes = (
        jax.ShapeDtypeStruct((B, E, H), jnp.float32),
        jax.ShapeDtypeStruct((B, E, 8), jnp.float32),
    )
    return pl.pallas_call(
        _edge_feat_kernel,
        grid=(B, NT),
        in_specs=[
            pl.BlockSpec((1, ET, 8), lambda b, t: (b, t, 0)),
            pl.BlockSpec((1, ET, EIN), lambda b, t: (b, t, 0)),
            pl.BlockSpec((1, N, 1), lambda b, t: (b, 0, 0)),
            pl.BlockSpec((1, N, 1), lambda b, t: (b, 0, 0)),
            pl.BlockSpec((1, NPT, 1), lambda b, t: (b, t, 0)),
            pl.BlockSpec((1, NPT, 1), lambda b, t: (b, t, 0)),
            _full((49, H)), _full((1, H)), _full((1, H)), _full((1, H)),
            _full((H + EIN, H)), _full((1, H)),
        ],
        out_specs=[pl.BlockSpec((1, ET, H), lambda b, t: (b, t, 0)),
                   pl.BlockSpec((1, ET, 8), lambda b, t: (b, t, 0))],
        out_shape=out_shapes,
    )(scalars, enb, mcol, ccol, mcol, ccol,
      params['Wef'], params['bef'].reshape(1, H),
      params['ln_ef_g'].reshape(1, H), params['ln_ef_b'].reshape(1, H),
      params['We'], params['be'].reshape(1, H))


def _msg(h_e, hv, hv_t, aux, w1, b1, w2, b2, w3, b3):
    eidx_f = aux[:, 0:1]
    mask_e = aux[:, 1:2]
    lane_n = lax.broadcasted_iota(jnp.int32, (ET, N), 1).astype(jnp.float32)
    oh = jnp.where(lane_n == eidx_f, 1.0, 0.0)
    e_iota = lax.broadcasted_iota(jnp.int32, (ET, NPT), 0)
    i_iota = lax.broadcasted_iota(jnp.int32, (ET, NPT), 1)
    dd = e_iota - K * i_iota
    ohi = jnp.where((dd >= 0) & (dd < K), 1.0, 0.0)
    a_i = _dot(hv_t, w1[:H, :])
    a_j = _dot(hv, w1[2 * H:, :])
    m = _dot(h_e, w1[H:2 * H, :])
    m = m + _dot(ohi, a_i)
    m = m + _dot(oh, a_j)
    m = jnp.maximum(m + b1, 0.0)
    m = jnp.maximum(_dot(m, w2) + b2, 0.0)
    m = _dot(m, w3) + b3
    return m, mask_e, ohi


def _edge_layer_kernel(he_ref, hv_ref, hvt_ref, aux_ref,
                       w1_ref, b1_ref, w2_ref, b2_ref, w3_ref, b3_ref,
                       l1g_ref, l1b_ref, wf1_ref, bf1_ref, wf2_ref, bf2_ref,
                       l2g_ref, l2b_ref, out_ref):
    h_e = he_ref[0]
    m, mask_e, _ = _msg(h_e, hv_ref[0], hvt_ref[0], aux_ref[0],
                        w1_ref[...], b1_ref[...], w2_ref[...], b2_ref[...],
                        w3_ref[...], b3_ref[...])
    h_e = _ln(h_e + m, l1g_ref[...], l1b_ref[...])
    ff = _dot(jnp.maximum(_dot(h_e, wf1_ref[...]) + bf1_ref[...], 0.0),
              wf2_ref[...]) + bf2_ref[...]
    out_ref[0] = _ln(h_e + ff, l2g_ref[...], l2b_ref[...]) * mask_e


def _node_layer_kernel(he_ref, hv_ref, hvt_ref, aux_ref,
                       w1_ref, b1_ref, w2_ref, b2_ref, w3_ref, b3_ref,
                       l1g_ref, l1b_ref, wf1_ref, bf1_ref, wf2_ref, bf2_ref,
                       l2g_ref, l2b_ref, mcolt_ref, out_ref):
    hv_t = hvt_ref[0]
    m, mask_e, ohi = _msg(he_ref[0], hv_ref[0], hv_t, aux_ref[0],
                          w1_ref[...], b1_ref[...], w2_ref[...], b2_ref[...],
                          w3_ref[...], b3_ref[...])
    m = m * mask_e
    dh = lax.dot_general(ohi, m, (((0,), (0,)), ((), ())),
                         preferred_element_type=jnp.float32) / float(K)
    hv_t = _ln(hv_t + dh, l1g_ref[...], l1b_ref[...])
    ff = _dot(jnp.maximum(_dot(hv_t, wf1_ref[...]) + bf1_ref[...], 0.0),
              wf2_ref[...]) + bf2_ref[...]
    out_ref[0] = _ln(hv_t + ff, l2g_ref[...], l2b_ref[...]) * mcolt_ref[0]


def _wspecs():
    return [_full((3 * H, H)), _full((1, H)), _full((H, H)), _full((1, H)),
            _full((H, H)), _full((1, H)), _full((1, H)), _full((1, H)),
            _full((H, 4 * H)), _full((1, 4 * H)), _full((4 * H, H)),
            _full((1, H)), _full((1, H)), _full((1, H))]


def _wargs(p):
    return (p['W1'], p['b1'].reshape(1, H), p['W2'], p['b2'].reshape(1, H),
            p['W3'], p['b3'].reshape(1, H),
            p['ln1_g'].reshape(1, H), p['ln1_b'].reshape(1, H),
            p['Wff1'], p['bff1'].reshape(1, 4 * H),
            p['Wff2'], p['bff2'].reshape(1, H),
            p['ln2_g'].reshape(1, H), p['ln2_b'].reshape(1, H))


def _run_mpnn(aux, he0, mcol, hv0, params):
    he = he0
    hv = hv0
    et_spec = pl.BlockSpec((1, ET, H), lambda b, t: (b, t, 0))
    hv_spec = pl.BlockSpec((1, N, H), lambda b, t: (b, 0, 0))
    hvt_spec = pl.BlockSpec((1, NPT, H), lambda b, t: (b, t, 0))
    aux_spec = pl.BlockSpec((1, ET, 8), lambda b, t: (b, t, 0))
    mcolt_spec = pl.BlockSpec((1, NPT, 1), lambda b, t: (b, t, 0))
    for layer in params['layers']:
        he = pl.pallas_call(
            _edge_layer_kernel,
            grid=(B, NT),
            in_specs=[et_spec, hv_spec, hvt_spec, aux_spec] + _wspecs(),
            out_specs=et_spec,
            out_shape=jax.ShapeDtypeStruct((B, E, H), jnp.float32),
        )(he, hv, hv, aux, *_wargs(layer['edge']))
        hv = pl.pallas_call(
            _node_layer_kernel,
            grid=(B, NT),
            in_specs=[et_spec, hv_spec, hvt_spec, aux_spec] + _wspecs()
                     + [mcolt_spec],
            out_specs=hvt_spec,
            out_shape=jax.ShapeDtypeStruct((B, N, H), jnp.float32),
        )(he, hv, hv, aux, *_wargs(layer['node']), mcol)
    return he, hv


def _sscore_kernel(hv_ref, ws_ref, bs_ref, out_ref):
    out_ref[0] = _dot(hv_ref[0], ws_ref[...]) + bs_ref[...]


def _run_sscore(hv, params):
    return pl.pallas_call(
        _sscore_kernel,
        grid=(B,),
        in_specs=[_batched((N, H)), _full((H, 1)), _full((1, 1))],
        out_specs=_batched((N, 1)),
        out_shape=jax.ShapeDtypeStruct((B, N, 1), jnp.float32),
    )(hv, params['W_s'], params['b_s'].reshape(1, 1))


def _etab_kernel(he_ref, wout_ref, bout_ref, etab_ref):
    etab_ref[0] = _dot(he_ref[0], wout_ref[...]) + bout_ref[...]


def _run_etab(he, params):
    return pl.pallas_call(
        _etab_kernel,
        grid=(B, NT),
        in_specs=[
            pl.BlockSpec((1, ET, H), lambda b, t: (b, t, 0)),
            _full((H, OUT)), _full((1, OUT)),
        ],
        out_specs=pl.BlockSpec((1, ET, OUT), lambda b, t: (b, t, 0)),
        out_shape=jax.ShapeDtypeStruct((B, E, OUT), jnp.float32),
    )(he, params['W_out'], params['b_out'].reshape(1, OUT))


def kernel(V_embed, E_embed, X, x_mask, chain_idx, params):
    Ca = X[:, :, 1, :]
    CaT = Ca.transpose(0, 2, 1)
    Xb = X[:, :, :3, :].reshape(B, 3 * N, 3)
    mcol = x_mask.reshape(B, N, 1)
    mrow = x_mask.reshape(B, 1, N)
    ccol = chain_idx.astype(jnp.float32).reshape(B, N, 1)

    eidx, dnb, gidx, hv0 = _run_features(Ca, CaT, Xb, mcol, mrow, V_embed, params)

    table = E_embed.reshape(B * N * N, EIN)
    enb = _run_sc_gather(table, gidx.reshape(B * E)).reshape(B, E, EIN)

    scalars = jnp.concatenate(
        [eidx.astype(jnp.float32).reshape(B, E, 1),
         dnb.reshape(B, E, 1),
         jnp.zeros((B, E, 6), jnp.float32)], axis=-1)
    he0, aux = _run_edge_features(scalars, enb, mcol, ccol, params)
    he, hv = _run_mpnn(aux, he0, mcol, hv0, params)
    sscore = _run_sscore(hv, params)
    etab = _run_etab(he, params)

    return (etab.reshape(B, N, K, OUT), eidx, sscore.reshape(B, N))


# speedup vs baseline: 5.1842x; 1.0007x over previous
import math
import jax
import jax.numpy as jnp
from jax import lax
from jax.experimental import pallas as pl
from jax.experimental.pallas import tpu as pltpu
from jax.experimental.pallas import tpu_sc as plsc

B, N, K, H = 8, 256, 30, 128
EIN = 128
OUT = 400
NUM_LAYERS = 3
E = N * K
NT = 4
ET = E // NT
NPT = N // NT

_NC, _NS = 2, 16
_NW = _NC * _NS
_ROWS_PER_W = (B * E) // _NW
_CHUNK = 128
_NCHUNK = _ROWS_PER_W // _CHUNK


def _ln(x, g, b, eps=1e-5):
    mu = jnp.mean(x, axis=-1, keepdims=True)
    var = jnp.mean((x - mu) ** 2, axis=-1, keepdims=True)
    return (x - mu) / jnp.sqrt(var + eps) * g + b


def _dot(a, b):
    return jnp.dot(a, b, preferred_element_type=jnp.float32)


def _full(shape):
    return pl.BlockSpec(shape, lambda *_: (0,) * len(shape))


def _batched(shape):
    return pl.BlockSpec((1,) + shape, lambda b, *_: (b,) + (0,) * len(shape))


def _tiled(shape):
    return pl.BlockSpec((1,) + shape, lambda b, t: (b, t) + (0,) * (len(shape) - 1))


def _feat_kernel(ca_ref, cat_ref, xb_ref, mcol_ref, mrow_ref, vembed_ref,
                 wvf_ref, bvf_ref, lnvg_ref, lnvb_ref, wv_ref, bv_ref,
                 eidx_ref, dnb_ref, gidx_ref, hv0_ref):
    b = pl.program_id(0)
    ca = ca_ref[0]
    cat = cat_ref[0]
    mcol = mcol_ref[0]
    mrow = mrow_ref[0]

    d2 = jnp.zeros((N, N), jnp.float32)
    for c in range(3):
        diff = ca[:, c:c + 1] - cat[c:c + 1, :]
        d2 = d2 + diff * diff
    dist = jnp.sqrt(d2 + 1e-6)
    mask2 = mcol * mrow
    work = dist + (1.0 - mask2) * 1e6

    lane = lax.broadcasted_iota(jnp.int32, (N, N), 1)
    idx_cols = []
    val_cols = []
    for _ in range(K):
        m = jnp.min(work, axis=1, keepdims=True)
        hit = work == m
        idx = jnp.min(jnp.where(hit, lane, N), axis=1, keepdims=True)
        idx_cols.append(idx)
        val_cols.append(m)
        work = jnp.where(lane == idx, 1e30, work)
    eidx = jnp.concatenate(idx_cols, axis=1)
    dnb = jnp.concatenate(val_cols, axis=1)
    eidx_ref[0] = eidx
    dnb_ref[0] = dnb
    gidx_ref[0] = eidx + (b * N + lax.broadcasted_iota(jnp.int32, (N, K), 0)) * N

    xb = xb_ref[0]
    dx = xb[1:, :] - xb[:-1, :]
    nrm = jnp.sqrt(jnp.sum(dx * dx, axis=1, keepdims=True))
    u = dx / (nrm + 1e-7)
    u2 = u[:-2, :]
    u1 = u[1:-1, :]
    u0 = u[2:, :]

    def cross(a, bb):
        return jnp.concatenate([
            a[:, 1:2] * bb[:, 2:3] - a[:, 2:3] * bb[:, 1:2],
            a[:, 2:3] * bb[:, 0:1] - a[:, 0:1] * bb[:, 2:3],
            a[:, 0:1] * bb[:, 1:2] - a[:, 1:2] * bb[:, 0:1],
        ], axis=1)

    def norm3(v):
        nn = jnp.sqrt(jnp.sum(v * v, axis=1, keepdims=True))
        return v / (nn + 1e-7)

    n2 = norm3(cross(u2, u1))
    n1 = norm3(cross(u1, u0))
    cosd = jnp.sum(n2 * n1, axis=1, keepdims=True)
    cosd = jnp.clip(cosd, -1.0 + 1e-7, 1.0 - 1e-7)
    sgn = jnp.sign(jnp.sum(u2 * n1, axis=1, keepdims=True))
    sind = sgn * jnp.sqrt(jnp.maximum(1.0 - cosd * cosd, 0.0))
    cosp = jnp.concatenate(
        [jnp.ones((1, 1), jnp.float32), cosd, jnp.ones((2, 1), jnp.float32)], axis=0)
    sinp = jnp.concatenate(
        [jnp.zeros((1, 1), jnp.float32), sind, jnp.zeros((2, 1), jnp.float32)], axis=0)

    i_row = lax.broadcasted_iota(jnp.int32, (N, 3 * N), 0)
    j_col = lax.broadcasted_iota(jnp.int32, (N, 3 * N), 1)
    cols = []
    for c in range(3):
        sel = jnp.where(j_col == 3 * i_row + c, 1.0, 0.0)
        cols.append(_dot(sel, cosp))
    for c in range(3):
        sel = jnp.where(j_col == 3 * i_row + c, 1.0, 0.0)
        cols.append(_dot(sel, sinp))
    vraw = jnp.concatenate(cols, axis=1)

    v = _ln(_dot(vraw, wvf_ref[...]) + bvf_ref[...], lnvg_ref[...], lnvb_ref[...])
    wv = wv_ref[...]
    hv0_ref[0] = (_dot(v, wv[:H, :]) + _dot(vembed_ref[0], wv[H:, :])
                  + bv_ref[...])


def _run_features(Ca, CaT, Xb, mcol, mrow, V_embed, params):
    out_shapes = (
        jax.ShapeDtypeStruct((B, N, K), jnp.int32),
        jax.ShapeDtypeStruct((B, N, K), jnp.float32),
        jax.ShapeDtypeStruct((B, N, K), jnp.int32),
        jax.ShapeDtypeStruct((B, N, H), jnp.float32),
    )
    return pl.pallas_call(
        _feat_kernel,
        grid=(B,),
        in_specs=[
            _batched((N, 3)), _batched((3, N)), _batched((3 * N, 3)),
            _batched((N, 1)), _batched((1, N)), _batched((N, EIN)),
            _full((6, H)), _full((1, H)), _full((1, H)), _full((1, H)),
            _full((H + EIN, H)), _full((1, H)),
        ],
        out_specs=[_batched((N, K)), _batched((N, K)), _batched((N, K)),
                   _batched((N, H))],
        out_shape=out_shapes,
    )(Ca, CaT, Xb, mcol, mrow, V_embed,
      params['Wvf'], params['bvf'].reshape(1, H),
      params['ln_vf_g'].reshape(1, H), params['ln_vf_b'].reshape(1, H),
      params['Wv'], params['bv'].reshape(1, H))


def _sc_gather_body(table_hbm, idx_hbm, out_hbm, idx_v, buf_v, sem):
    wid = lax.axis_index("s") * _NC + lax.axis_index("c")
    base = wid * _ROWS_PER_W
    table = table_hbm.reshape(B * N * N, EIN)
    pltpu.sync_copy(idx_hbm.at[pl.ds(base, _ROWS_PER_W)], idx_v)
    for j in range(_NCHUNK):
        pltpu.async_copy(
            table.at[idx_v.at[pl.ds(j * _CHUNK, _CHUNK)]], buf_v, sem).wait()
        pltpu.sync_copy(buf_v, out_hbm.at[pl.ds(base + j * _CHUNK, _CHUNK)])


def _run_sc_gather(table_flat, gidx_flat):
    import functools
    mesh = plsc.VectorSubcoreMesh(core_axis_name="c", subcore_axis_name="s")
    k = functools.partial(
        pl.kernel,
        mesh=mesh,
        out_type=jax.ShapeDtypeStruct((B * E, EIN), jnp.float32),
        scratch_types=[
            pltpu.VMEM((_ROWS_PER_W,), jnp.int32),
            pltpu.VMEM((_CHUNK, EIN), jnp.float32),
            pltpu.SemaphoreType.DMA,
        ],
    )(_sc_gather_body)
    return k(table_flat, gidx_flat)


def _edge_feat_kernel(sc_ref, enb_ref, mcol_ref, ccol_ref, mcolt_ref, ccolt_ref,
                      wef_ref, bef_ref, lneg_ref, lneb_ref, we_ref, be_ref,
                      he0_ref, aux_ref):
    t = pl.program_id(1)
    eidx_f = sc_ref[0][:, 0:1]
    dnb = sc_ref[0][:, 1:2]
    mcol = mcol_ref[0]
    ccol = ccol_ref[0]

    lane_n = lax.broadcasted_iota(jnp.int32, (ET, N), 1).astype(jnp.float32)
    oh = jnp.where(lane_n == eidx_f, 1.0, 0.0)
    e_iota = lax.broadcasted_iota(jnp.int32, (ET, NPT), 0)
    i_iota = lax.broadcasted_iota(jnp.int32, (ET, NPT), 1)
    dd = e_iota - K * i_iota
    ohi = jnp.where((dd >= 0) & (dd < K), 1.0, 0.0)

    node0 = t * NPT
    iota_np = lax.broadcasted_iota(jnp.int32, (NPT, 1), 0).astype(jnp.float32)
    row_i = _dot(ohi, iota_np) + lax.convert_element_type(node0, jnp.float32)
    mcol_t = mcolt_ref[0]
    ccol_t = ccolt_ref[0]
    mask_i = _dot(ohi, mcol_t)
    mask_j = _dot(oh, mcol)
    mask_e = mask_i * mask_j
    ci_i = _dot(ohi, ccol_t)
    ci_j = _dot(oh, ccol)
    same = jnp.where(ci_i == ci_j, 1.0, 0.0)
    rel = (eidx_f - row_i) * same

    kf = lax.broadcasted_iota(jnp.int32, (ET, 16), 1).astype(jnp.float32)
    freq = jnp.exp(kf * 2.0 * (-math.log(10000.0) / 32.0))
    ang = rel * freq
    mu = 2.0 + kf * (20.0 / 15.0)
    sigma = (22.0 - 2.0) / 16.0
    rbf = jnp.exp(-(((dnb - mu) / sigma) ** 2))
    wef = wef_ref[...]
    pre = (_dot(jnp.cos(ang), wef[0:16, :]) + _dot(jnp.sin(ang), wef[16:32, :])
           + _dot(rbf, wef[32:48, :]) + same * wef[48:49, :] + bef_ref[...])
    e_emb = _ln(pre, lneg_ref[...], lneb_ref[...])
    we = we_ref[...]
    he0_ref[0] = _dot(e_emb, we[:H, :]) + _dot(enb_ref[0], we[H:, :]) + be_ref[...]

    aux = jnp.concatenate(
        [eidx_f, mask_e, row_i,
         jnp.zeros((ET, 5), jnp.float32)], axis=1)
    aux_ref[0] = aux


def _run_edge_features(scalars, enb, mcol, ccol, params):
    out_shapes = (
        jax.ShapeDtypeStruct((B, E, H), jnp.float32),
        jax.ShapeDtypeStruct((B, E, 8), jnp.float32),
    )
    return pl.pallas_call(
        _edge_feat_kernel,
        grid=(B, NT),
        in_specs=[
            pl.BlockSpec((1, ET, 8), lambda b, t: (b, t, 0)),
            pl.BlockSpec((1, ET, EIN), lambda b, t: (b, t, 0)),
            pl.BlockSpec((1, N, 1), lambda b, t: (b, 0, 0)),
            pl.BlockSpec((1, N, 1), lambda b, t: (b, 0, 0)),
            pl.BlockSpec((1, NPT, 1), lambda b, t: (b, t, 0)),
            pl.BlockSpec((1, NPT, 1), lambda b, t: (b, t, 0)),
            _full((49, H)), _full((1, H)), _full((1, H)), _full((1, H)),
            _full((H + EIN, H)), _full((1, H)),
        ],
        out_specs=[pl.BlockSpec((1, ET, H), lambda b, t: (b, t, 0)),
                   pl.BlockSpec((1, ET, 8), lambda b, t: (b, t, 0))],
        out_shape=out_shapes,
    )(scalars, enb, mcol, ccol, mcol, ccol,
      params['Wef'], params['bef'].reshape(1, H),
      params['ln_ef_g'].reshape(1, H), params['ln_ef_b'].reshape(1, H),
      params['We'], params['be'].reshape(1, H))


def _msg(h_e, hv, hv_t, aux, w1, b1, w2, b2, w3, b3):
    eidx_f = aux[:, 0:1]
    mask_e = aux[:, 1:2]
    lane_n = lax.broadcasted_iota(jnp.int32, (ET, N), 1).astype(jnp.float32)
    oh = jnp.where(lane_n == eidx_f, 1.0, 0.0)
    e_iota = lax.broadcasted_iota(jnp.int32, (ET, NPT), 0)
    i_iota = lax.broadcasted_iota(jnp.int32, (ET, NPT), 1)
    dd = e_iota - K * i_iota
    ohi = jnp.where((dd >= 0) & (dd < K), 1.0, 0.0)
    a_i = _dot(hv_t, w1[:H, :])
    a_j = _dot(hv, w1[2 * H:, :])
    m = _dot(h_e, w1[H:2 * H, :])
    m = m + _dot(ohi, a_i)
    m = m + _dot(oh, a_j)
    m = jnp.maximum(m + b1, 0.0)
    m = jnp.maximum(_dot(m, w2) + b2, 0.0)
    m = _dot(m, w3) + b3
    return m, mask_e, ohi


def _edge_layer_kernel(he_ref, hv_ref, hvt_ref, aux_ref,
                       w1_ref, b1_ref, w2_ref, b2_ref, w3_ref, b3_ref,
                       l1g_ref, l1b_ref, wf1_ref, bf1_ref, wf2_ref, bf2_ref,
                       l2g_ref, l2b_ref, out_ref):
    h_e = he_ref[0]
    m, mask_e, _ = _msg(h_e, hv_ref[0], hvt_ref[0], aux_ref[0],
                        w1_ref[...], b1_ref[...], w2_ref[...], b2_ref[...],
                        w3_ref[...], b3_ref[...])
    h_e = _ln(h_e + m, l1g_ref[...], l1b_ref[...])
    ff = _dot(jnp.maximum(_dot(h_e, wf1_ref[...]) + bf1_ref[...], 0.0),
              wf2_ref[...]) + bf2_ref[...]
    out_ref[0] = _ln(h_e + ff, l2g_ref[...], l2b_ref[...]) * mask_e


def _node_layer_kernel(he_ref, hv_ref, hvt_ref, aux_ref,
                       w1_ref, b1_ref, w2_ref, b2_ref, w3_ref, b3_ref,
                       l1g_ref, l1b_ref, wf1_ref, bf1_ref, wf2_ref, bf2_ref,
                       l2g_ref, l2b_ref, mcolt_ref, out_ref):
    hv_t = hvt_ref[0]
    m, mask_e, ohi = _msg(he_ref[0], hv_ref[0], hv_t, aux_ref[0],
                          w1_ref[...], b1_ref[...], w2_ref[...], b2_ref[...],
                          w3_ref[...], b3_ref[...])
    m = m * mask_e
    dh = lax.dot_general(ohi, m, (((0,), (0,)), ((), ())),
                         preferred_element_type=jnp.float32) / float(K)
    hv_t = _ln(hv_t + dh, l1g_ref[...], l1b_ref[...])
    ff = _dot(jnp.maximum(_dot(hv_t, wf1_ref[...]) + bf1_ref[...], 0.0),
              wf2_ref[...]) + bf2_ref[...]
    out_ref[0] = _ln(hv_t + ff, l2g_ref[...], l2b_ref[...]) * mcolt_ref[0]


def _wspecs():
    return [_full((3 * H, H)), _full((1, H)), _full((H, H)), _full((1, H)),
            _full((H, H)), _full((1, H)), _full((1, H)), _full((1, H)),
            _full((H, 4 * H)), _full((1, 4 * H)), _full((4 * H, H)),
            _full((1, H)), _full((1, H)), _full((1, H))]


def _wargs(p):
    return (p['W1'], p['b1'].reshape(1, H), p['W2'], p['b2'].reshape(1, H),
            p['W3'], p['b3'].reshape(1, H),
            p['ln1_g'].reshape(1, H), p['ln1_b'].reshape(1, H),
            p['Wff1'], p['bff1'].reshape(1, 4 * H),
            p['Wff2'], p['bff2'].reshape(1, H),
            p['ln2_g'].reshape(1, H), p['ln2_b'].reshape(1, H))


def _run_mpnn(aux, he0, mcol, hv0, params):
    he = he0
    hv = hv0
    et_spec = pl.BlockSpec((1, ET, H), lambda b, t: (b, t, 0))
    hv_spec = pl.BlockSpec((1, N, H), lambda b, t: (b, 0, 0))
    hvt_spec = pl.BlockSpec((1, NPT, H), lambda b, t: (b, t, 0))
    aux_spec = pl.BlockSpec((1, ET, 8), lambda b, t: (b, t, 0))
    mcolt_spec = pl.BlockSpec((1, NPT, 1), lambda b, t: (b, t, 0))
    for layer in params['layers']:
        he = pl.pallas_call(
            _edge_layer_kernel,
            grid=(B, NT),
            in_specs=[et_spec, hv_spec, hvt_spec, aux_spec] + _wspecs(),
            out_specs=et_spec,
            out_shape=jax.ShapeDtypeStruct((B, E, H), jnp.float32),
        )(he, hv, hv, aux, *_wargs(layer['edge']))
        hv = pl.pallas_call(
            _node_layer_kernel,
            grid=(B, NT),
            in_specs=[et_spec, hv_spec, hvt_spec, aux_spec] + _wspecs()
                     + [mcolt_spec],
            out_specs=hvt_spec,
            out_shape=jax.ShapeDtypeStruct((B, N, H), jnp.float32),
        )(he, hv, hv, aux, *_wargs(layer['node']), mcol)
    return he, hv


def _sscore_kernel(hv_ref, ws_ref, bs_ref, out_ref):
    out_ref[0] = _dot(hv_ref[0], ws_ref[...]) + bs_ref[...]


def _run_sscore(hv, params):
    return pl.pallas_call(
        _sscore_kernel,
        grid=(B,),
        in_specs=[_batched((N, H)), _full((H, 1)), _full((1, 1))],
        out_specs=_batched((N, 1)),
        out_shape=jax.ShapeDtypeStruct((B, N, 1), jnp.float32),
    )(hv, params['W_s'], params['b_s'].reshape(1, 1))


def _etab_kernel(he_ref, wout_ref, bout_ref, etab_ref):
    etab_ref[0] = _dot(he_ref[0], wout_ref[...]) + bout_ref[...]


def _run_etab(he, params):
    return pl.pallas_call(
        _etab_kernel,
        grid=(B, NT),
        in_specs=[
            pl.BlockSpec((1, ET, H), lambda b, t: (b, t, 0)),
            _full((H, OUT)), _full((1, OUT)),
        ],
        out_specs=pl.BlockSpec((1, ET, OUT), lambda b, t: (b, t, 0)),
        out_shape=jax.ShapeDtypeStruct((B, E, OUT), jnp.float32),
    )(he, params['W_out'], params['b_out'].reshape(1, OUT))


def kernel(V_embed, E_embed, X, x_mask, chain_idx, params):
    Ca = X[:, :, 1, :]
    CaT = Ca.transpose(0, 2, 1)
    Xb = X[:, :, :3, :].reshape(B, 3 * N, 3)
    mcol = x_mask.reshape(B, N, 1)
    mrow = x_mask.reshape(B, 1, N)
    ccol = chain_idx.astype(jnp.float32).reshape(B, N, 1)

    eidx, dnb, gidx, hv0 = _run_features(Ca, CaT, Xb, mcol, mrow, V_embed, params)

    enb = _run_sc_gather(E_embed, gidx.reshape(B * E)).reshape(B, E, EIN)

    scalars = jnp.concatenate(
        [eidx.astype(jnp.float32).reshape(B, E, 1),
         dnb.reshape(B, E, 1),
         jnp.zeros((B, E, 6), jnp.float32)], axis=-1)
    he0, aux = _run_edge_features(scalars, enb, mcol, ccol, params)
    he, hv = _run_mpnn(aux, he0, mcol, hv0, params)
    sscore = _run_sscore(hv, params)
    etab = _run_etab(he, params)

    return (etab.reshape(B, N, K, OUT), eidx, sscore.reshape(B, N))


# speedup vs baseline: 5.2775x; 1.0180x over previous
import math
import jax
import jax.numpy as jnp
from jax import lax
from jax.experimental import pallas as pl
from jax.experimental.pallas import tpu as pltpu
from jax.experimental.pallas import tpu_sc as plsc

B, N, K, H = 8, 256, 30, 128
EIN = 128
OUT = 400
NUM_LAYERS = 3
E = N * K
NT = 4
ET = E // NT
NPT = N // NT

_NC, _NS = 2, 16
_NW = _NC * _NS
_ROWS_PER_W = (B * E) // _NW
_CHUNK = 128
_NCHUNK = _ROWS_PER_W // _CHUNK


def _ln(x, g, b, eps=1e-5):
    mu = jnp.mean(x, axis=-1, keepdims=True)
    var = jnp.mean((x - mu) ** 2, axis=-1, keepdims=True)
    return (x - mu) / jnp.sqrt(var + eps) * g + b


def _dot(a, b):
    return jnp.dot(a, b, preferred_element_type=jnp.float32)


def _full(shape):
    return pl.BlockSpec(shape, lambda *_: (0,) * len(shape))


def _batched(shape):
    return pl.BlockSpec((1,) + shape, lambda b, *_: (b,) + (0,) * len(shape))


def _tiled(shape):
    return pl.BlockSpec((1,) + shape, lambda b, t: (b, t) + (0,) * (len(shape) - 1))


def _feat_kernel(ca_ref, cat_ref, xb_ref, mcol_ref, mrow_ref, vembed_ref,
                 wvf_ref, bvf_ref, lnvg_ref, lnvb_ref, wv_ref, bv_ref,
                 eidx_ref, dnb_ref, gidx_ref, hv0_ref):
    b = pl.program_id(0)
    ca = ca_ref[0]
    cat = cat_ref[0]
    mcol = mcol_ref[0]
    mrow = mrow_ref[0]

    d2 = jnp.zeros((N, N), jnp.float32)
    for c in range(3):
        diff = ca[:, c:c + 1] - cat[c:c + 1, :]
        d2 = d2 + diff * diff
    dist = jnp.sqrt(d2 + 1e-6)
    mask2 = mcol * mrow
    work = dist + (1.0 - mask2) * 1e6

    lane = lax.broadcasted_iota(jnp.int32, (N, N), 1)
    idx_cols = []
    val_cols = []
    for _ in range(K):
        m = jnp.min(work, axis=1, keepdims=True)
        hit = work == m
        idx = jnp.min(jnp.where(hit, lane, N), axis=1, keepdims=True)
        idx_cols.append(idx)
        val_cols.append(m)
        work = jnp.where(lane == idx, 1e30, work)
    eidx = jnp.concatenate(idx_cols, axis=1)
    dnb = jnp.concatenate(val_cols, axis=1)
    eidx_ref[0] = eidx
    dnb_ref[0] = dnb
    gidx_ref[0] = eidx + (b * N + lax.broadcasted_iota(jnp.int32, (N, K), 0)) * N

    xb = xb_ref[0]
    dx = xb[1:, :] - xb[:-1, :]
    nrm = jnp.sqrt(jnp.sum(dx * dx, axis=1, keepdims=True))
    u = dx / (nrm + 1e-7)
    u2 = u[:-2, :]
    u1 = u[1:-1, :]
    u0 = u[2:, :]

    def cross(a, bb):
        return jnp.concatenate([
            a[:, 1:2] * bb[:, 2:3] - a[:, 2:3] * bb[:, 1:2],
            a[:, 2:3] * bb[:, 0:1] - a[:, 0:1] * bb[:, 2:3],
            a[:, 0:1] * bb[:, 1:2] - a[:, 1:2] * bb[:, 0:1],
        ], axis=1)

    def norm3(v):
        nn = jnp.sqrt(jnp.sum(v * v, axis=1, keepdims=True))
        return v / (nn + 1e-7)

    n2 = norm3(cross(u2, u1))
    n1 = norm3(cross(u1, u0))
    cosd = jnp.sum(n2 * n1, axis=1, keepdims=True)
    cosd = jnp.clip(cosd, -1.0 + 1e-7, 1.0 - 1e-7)
    sgn = jnp.sign(jnp.sum(u2 * n1, axis=1, keepdims=True))
    sind = sgn * jnp.sqrt(jnp.maximum(1.0 - cosd * cosd, 0.0))
    cosp = jnp.concatenate(
        [jnp.ones((1, 1), jnp.float32), cosd, jnp.ones((2, 1), jnp.float32)], axis=0)
    sinp = jnp.concatenate(
        [jnp.zeros((1, 1), jnp.float32), sind, jnp.zeros((2, 1), jnp.float32)], axis=0)

    i_row = lax.broadcasted_iota(jnp.int32, (N, 3 * N), 0)
    j_col = lax.broadcasted_iota(jnp.int32, (N, 3 * N), 1)
    cols = []
    for c in range(3):
        sel = jnp.where(j_col == 3 * i_row + c, 1.0, 0.0)
        cols.append(_dot(sel, cosp))
    for c in range(3):
        sel = jnp.where(j_col == 3 * i_row + c, 1.0, 0.0)
        cols.append(_dot(sel, sinp))
    vraw = jnp.concatenate(cols, axis=1)

    v = _ln(_dot(vraw, wvf_ref[...]) + bvf_ref[...], lnvg_ref[...], lnvb_ref[...])
    wv = wv_ref[...]
    hv0_ref[0] = (_dot(v, wv[:H, :]) + _dot(vembed_ref[0], wv[H:, :])
                  + bv_ref[...])


def _run_features(Ca, CaT, Xb, mcol, mrow, V_embed, params):
    out_shapes = (
        jax.ShapeDtypeStruct((B, N, K), jnp.int32),
        jax.ShapeDtypeStruct((B, N, K), jnp.float32),
        jax.ShapeDtypeStruct((B, N, K), jnp.int32),
        jax.ShapeDtypeStruct((B, N, H), jnp.float32),
    )
    return pl.pallas_call(
        _feat_kernel,
        grid=(B,),
        in_specs=[
            _batched((N, 3)), _batched((3, N)), _batched((3 * N, 3)),
            _batched((N, 1)), _batched((1, N)), _batched((N, EIN)),
            _full((6, H)), _full((1, H)), _full((1, H)), _full((1, H)),
            _full((H + EIN, H)), _full((1, H)),
        ],
        out_specs=[_batched((N, K)), _batched((N, K)), _batched((N, K)),
                   _batched((N, H))],
        out_shape=out_shapes,
    )(Ca, CaT, Xb, mcol, mrow, V_embed,
      params['Wvf'], params['bvf'].reshape(1, H),
      params['ln_vf_g'].reshape(1, H), params['ln_vf_b'].reshape(1, H),
      params['Wv'], params['bv'].reshape(1, H))


def _sc_gather_body(table_hbm, idx_hbm, out_hbm, idx_v, buf_v, sem):
    wid = lax.axis_index("s") * _NC + lax.axis_index("c")
    base = wid * _ROWS_PER_W
    table = table_hbm.reshape(B * N * N, EIN)
    pltpu.sync_copy(idx_hbm.at[pl.ds(base, _ROWS_PER_W)], idx_v)
    for j in range(_NCHUNK):
        pltpu.async_copy(
            table.at[idx_v.at[pl.ds(j * _CHUNK, _CHUNK)]], buf_v, sem).wait()
        pltpu.sync_copy(buf_v, out_hbm.at[pl.ds(base + j * _CHUNK, _CHUNK)])


def _run_sc_gather(table_flat, gidx_flat):
    import functools
    mesh = plsc.VectorSubcoreMesh(core_axis_name="c", subcore_axis_name="s")
    k = functools.partial(
        pl.kernel,
        mesh=mesh,
        out_type=jax.ShapeDtypeStruct((B * E, EIN), jnp.float32),
        scratch_types=[
            pltpu.VMEM((_ROWS_PER_W,), jnp.int32),
            pltpu.VMEM((_CHUNK, EIN), jnp.float32),
            pltpu.SemaphoreType.DMA,
        ],
    )(_sc_gather_body)
    return k(table_flat, gidx_flat)


def _edge_feat_kernel(sc_ref, enb_ref, mcol_ref, ccol_ref, mcolt_ref, ccolt_ref,
                      wef_ref, bef_ref, lneg_ref, lneb_ref, we_ref, be_ref,
                      he0_ref, aux_ref):
    t = pl.program_id(1)
    eidx_f = sc_ref[0][:, 0:1]
    dnb = sc_ref[0][:, 1:2]
    mcol = mcol_ref[0]
    ccol = ccol_ref[0]

    lane_n = lax.broadcasted_iota(jnp.int32, (ET, N), 1).astype(jnp.float32)
    oh = jnp.where(lane_n == eidx_f, 1.0, 0.0)
    e_iota = lax.broadcasted_iota(jnp.int32, (ET, NPT), 0)
    i_iota = lax.broadcasted_iota(jnp.int32, (ET, NPT), 1)
    dd = e_iota - K * i_iota
    ohi = jnp.where((dd >= 0) & (dd < K), 1.0, 0.0)

    node0 = t * NPT
    iota_np = lax.broadcasted_iota(jnp.int32, (NPT, 1), 0).astype(jnp.float32)
    row_i = _dot(ohi, iota_np) + lax.convert_element_type(node0, jnp.float32)
    mcol_t = mcolt_ref[0]
    ccol_t = ccolt_ref[0]
    mask_i = _dot(ohi, mcol_t)
    mask_j = _dot(oh, mcol)
    mask_e = mask_i * mask_j
    ci_i = _dot(ohi, ccol_t)
    ci_j = _dot(oh, ccol)
    same = jnp.where(ci_i == ci_j, 1.0, 0.0)
    rel = (eidx_f - row_i) * same

    kf = lax.broadcasted_iota(jnp.int32, (ET, 16), 1).astype(jnp.float32)
    freq = jnp.exp(kf * 2.0 * (-math.log(10000.0) / 32.0))
    ang = rel * freq
    mu = 2.0 + kf * (20.0 / 15.0)
    sigma = (22.0 - 2.0) / 16.0
    rbf = jnp.exp(-(((dnb - mu) / sigma) ** 2))
    wef = wef_ref[...]
    pre = (_dot(jnp.cos(ang), wef[0:16, :]) + _dot(jnp.sin(ang), wef[16:32, :])
           + _dot(rbf, wef[32:48, :]) + same * wef[48:49, :] + bef_ref[...])
    e_emb = _ln(pre, lneg_ref[...], lneb_ref[...])
    we = we_ref[...]
    he0_ref[0] = _dot(e_emb, we[:H, :]) + _dot(enb_ref[0], we[H:, :]) + be_ref[...]

    aux = jnp.concatenate(
        [eidx_f, mask_e, row_i,
         jnp.zeros((ET, 5), jnp.float32)], axis=1)
    aux_ref[0] = aux


def _run_edge_features(scalars, enb, mcol, ccol, params):
    out_shapes = (
        jax.ShapeDtypeStruct((B, E, H), jnp.float32),
        jax.ShapeDtypeStruct((B, E, 8), jnp.float32),
    )
    return pl.pallas_call(
        _edge_feat_kernel,
        grid=(B, NT),
        in_specs=[
            pl.BlockSpec((1, ET, 8), lambda b, t: (b, t, 0)),
            pl.BlockSpec((1, ET, EIN), lambda b, t: (b, t, 0)),
            pl.BlockSpec((1, N, 1), lambda b, t: (b, 0, 0)),
            pl.BlockSpec((1, N, 1), lambda b, t: (b, 0, 0)),
            pl.BlockSpec((1, NPT, 1), lambda b, t: (b, t, 0)),
            pl.BlockSpec((1, NPT, 1), lambda b, t: (b, t, 0)),
            _full((49, H)), _full((1, H)), _full((1, H)), _full((1, H)),
            _full((H + EIN, H)), _full((1, H)),
        ],
        out_specs=[pl.BlockSpec((1, ET, H), lambda b, t: (b, t, 0)),
                   pl.BlockSpec((1, ET, 8), lambda b, t: (b, t, 0))],
        out_shape=out_shapes,
    )(scalars, enb, mcol, ccol, mcol, ccol,
      params['Wef'], params['bef'].reshape(1, H),
      params['ln_ef_g'].reshape(1, H), params['ln_ef_b'].reshape(1, H),
      params['We'], params['be'].reshape(1, H))


def _msg(h_e, hv, hv_t, aux, w1, b1, w2, b2, w3, b3):
    eidx_f = aux[:, 0:1]
    mask_e = aux[:, 1:2]
    lane_n = lax.broadcasted_iota(jnp.int32, (ET, N), 1).astype(jnp.float32)
    oh = jnp.where(lane_n == eidx_f, 1.0, 0.0)
    e_iota = lax.broadcasted_iota(jnp.int32, (ET, NPT), 0)
    i_iota = lax.broadcasted_iota(jnp.int32, (ET, NPT), 1)
    dd = e_iota - K * i_iota
    ohi = jnp.where((dd >= 0) & (dd < K), 1.0, 0.0)
    a_i = _dot(hv_t, w1[:H, :])
    a_j = _dot(hv, w1[2 * H:, :])
    m = _dot(h_e, w1[H:2 * H, :])
    m = m + _dot(ohi, a_i)
    m = m + _dot(oh, a_j)
    m = jnp.maximum(m + b1, 0.0)
    m = jnp.maximum(_dot(m, w2) + b2, 0.0)
    m = _dot(m, w3) + b3
    return m, mask_e, ohi


def _layer_kernel(he_ref, hv_ref, hvt_ref, aux_ref, mcolt_ref,
                  ew1_ref, eb1_ref, ew2_ref, eb2_ref, ew3_ref, eb3_ref,
                  el1g_ref, el1b_ref, ewf1_ref, ebf1_ref, ewf2_ref, ebf2_ref,
                  el2g_ref, el2b_ref,
                  nw1_ref, nb1_ref, nw2_ref, nb2_ref, nw3_ref, nb3_ref,
                  nl1g_ref, nl1b_ref, nwf1_ref, nbf1_ref, nwf2_ref, nbf2_ref,
                  nl2g_ref, nl2b_ref,
                  heo_ref, hvo_ref):
    h_e = he_ref[0]
    hv = hv_ref[0]
    hv_t = hvt_ref[0]
    aux = aux_ref[0]

    m, mask_e, _ = _msg(h_e, hv, hv_t, aux,
                        ew1_ref[...], eb1_ref[...], ew2_ref[...], eb2_ref[...],
                        ew3_ref[...], eb3_ref[...])
    h_e = _ln(h_e + m, el1g_ref[...], el1b_ref[...])
    ff = _dot(jnp.maximum(_dot(h_e, ewf1_ref[...]) + ebf1_ref[...], 0.0),
              ewf2_ref[...]) + ebf2_ref[...]
    h_e = _ln(h_e + ff, el2g_ref[...], el2b_ref[...]) * mask_e
    heo_ref[0] = h_e

    m, mask_e, ohi = _msg(h_e, hv, hv_t, aux,
                          nw1_ref[...], nb1_ref[...], nw2_ref[...], nb2_ref[...],
                          nw3_ref[...], nb3_ref[...])
    m = m * mask_e
    dh = lax.dot_general(ohi, m, (((0,), (0,)), ((), ())),
                         preferred_element_type=jnp.float32) / float(K)
    hv_t = _ln(hv_t + dh, nl1g_ref[...], nl1b_ref[...])
    ff = _dot(jnp.maximum(_dot(hv_t, nwf1_ref[...]) + nbf1_ref[...], 0.0),
              nwf2_ref[...]) + nbf2_ref[...]
    hvo_ref[0] = _ln(hv_t + ff, nl2g_ref[...], nl2b_ref[...]) * mcolt_ref[0]


def _wspecs():
    return [_full((3 * H, H)), _full((1, H)), _full((H, H)), _full((1, H)),
            _full((H, H)), _full((1, H)), _full((1, H)), _full((1, H)),
            _full((H, 4 * H)), _full((1, 4 * H)), _full((4 * H, H)),
            _full((1, H)), _full((1, H)), _full((1, H))]


def _wargs(p):
    return (p['W1'], p['b1'].reshape(1, H), p['W2'], p['b2'].reshape(1, H),
            p['W3'], p['b3'].reshape(1, H),
            p['ln1_g'].reshape(1, H), p['ln1_b'].reshape(1, H),
            p['Wff1'], p['bff1'].reshape(1, 4 * H),
            p['Wff2'], p['bff2'].reshape(1, H),
            p['ln2_g'].reshape(1, H), p['ln2_b'].reshape(1, H))


def _run_mpnn(aux, he0, mcol, hv0, params):
    he = he0
    hv = hv0
    et_spec = pl.BlockSpec((1, ET, H), lambda b, t: (b, t, 0))
    hv_spec = pl.BlockSpec((1, N, H), lambda b, t: (b, 0, 0))
    hvt_spec = pl.BlockSpec((1, NPT, H), lambda b, t: (b, t, 0))
    aux_spec = pl.BlockSpec((1, ET, 8), lambda b, t: (b, t, 0))
    mcolt_spec = pl.BlockSpec((1, NPT, 1), lambda b, t: (b, t, 0))
    for layer in params['layers']:
        he, hv = pl.pallas_call(
            _layer_kernel,
            grid=(B, NT),
            in_specs=[et_spec, hv_spec, hvt_spec, aux_spec, mcolt_spec]
                     + _wspecs() + _wspecs(),
            out_specs=[et_spec, hvt_spec],
            out_shape=(jax.ShapeDtypeStruct((B, E, H), jnp.float32),
                       jax.ShapeDtypeStruct((B, N, H), jnp.float32)),
        )(he, hv, hv, aux, mcol, *_wargs(layer['edge']), *_wargs(layer['node']))
    return he, hv


def _sscore_kernel(hv_ref, ws_ref, bs_ref, out_ref):
    out_ref[0] = _dot(hv_ref[0], ws_ref[...]) + bs_ref[...]


def _run_sscore(hv, params):
    return pl.pallas_call(
        _sscore_kernel,
        grid=(B,),
        in_specs=[_batched((N, H)), _full((H, 1)), _full((1, 1))],
        out_specs=_batched((N, 1)),
        out_shape=jax.ShapeDtypeStruct((B, N, 1), jnp.float32),
    )(hv, params['W_s'], params['b_s'].reshape(1, 1))


def _etab_kernel(he_ref, wout_ref, bout_ref, etab_ref):
    etab_ref[0] = _dot(he_ref[0], wout_ref[...]) + bout_ref[...]


def _run_etab(he, params):
    return pl.pallas_call(
        _etab_kernel,
        grid=(B, NT),
        in_specs=[
            pl.BlockSpec((1, ET, H), lambda b, t: (b, t, 0)),
            _full((H, OUT)), _full((1, OUT)),
        ],
        out_specs=pl.BlockSpec((1, ET, OUT), lambda b, t: (b, t, 0)),
        out_shape=jax.ShapeDtypeStruct((B, E, OUT), jnp.float32),
    )(he, params['W_out'], params['b_out'].reshape(1, OUT))


def kernel(V_embed, E_embed, X, x_mask, chain_idx, params):
    Ca = X[:, :, 1, :]
    CaT = Ca.transpose(0, 2, 1)
    Xb = X[:, :, :3, :].reshape(B, 3 * N, 3)
    mcol = x_mask.reshape(B, N, 1)
    mrow = x_mask.reshape(B, 1, N)
    ccol = chain_idx.astype(jnp.float32).reshape(B, N, 1)

    eidx, dnb, gidx, hv0 = _run_features(Ca, CaT, Xb, mcol, mrow, V_embed, params)

    enb = _run_sc_gather(E_embed, gidx.reshape(B * E)).reshape(B, E, EIN)

    scalars = jnp.concatenate(
        [eidx.astype(jnp.float32).reshape(B, E, 1),
         dnb.reshape(B, E, 1),
         jnp.zeros((B, E, 6), jnp.float32)], axis=-1)
    he0, aux = _run_edge_features(scalars, enb, mcol, ccol, params)
    he, hv = _run_mpnn(aux, he0, mcol, hv0, params)
    sscore = _run_sscore(hv, params)
    etab = _run_etab(he, params)

    return (etab.reshape(B, N, K, OUT), eidx, sscore.reshape(B, N))
